# Initial kernel scaffold; baseline (speedup 1.0000x reference)
import jax
import jax.numpy as jnp
from jax.experimental import pallas as pl


def kernel(x, emb, W, b):
    raise NotImplementedError("write your pallas kernel here")


# speedup vs baseline: 7.3728x; 7.3728x over previous
import functools

import jax
import jax.numpy as jnp
from jax import lax
from jax.experimental import pallas as pl
from jax.experimental.pallas import tpu as pltpu
from jax.experimental.pallas import tpu_sc as plsc

VOCAB = 100000
EMB_DIM = 300
NUM_CLASSES = 2
BATCH = 4096
SEQ = 200

DP = 16
NC, NS = 2, 16
NW = NC * NS
ROWS_PER_W = BATCH // NW
G_ROWS = 8
NGROUPS = ROWS_PER_W // G_ROWS
TOK_CHUNK = 100
STREAMS_PER_GROUP = G_ROWS * SEQ // TOK_CHUNK
CHUNKS_PER_W = ROWS_PER_W * SEQ // TOK_CHUNK

PROJ_BLK = 2000


def _proj_body(emb_ref, wt_ref, out_ref):
    out_ref[...] = jnp.dot(emb_ref[...], wt_ref[...],
                           preferred_element_type=jnp.float32)


def _project(emb, wt):
    return pl.pallas_call(
        _proj_body,
        grid=(VOCAB // PROJ_BLK,),
        in_specs=[
            pl.BlockSpec((PROJ_BLK, EMB_DIM), lambda i: (i, 0)),
            pl.BlockSpec((EMB_DIM, DP), lambda i: (0, 0)),
        ],
        out_specs=pl.BlockSpec((PROJ_BLK, DP), lambda i: (i, 0)),
        out_shape=jax.ShapeDtypeStruct((VOCAB, DP), jnp.float32),
    )(emb, wt)


def _pool_body(proj_hbm, x2_hbm, bvec_hbm, out_hbm,
               idx_v, buf0, buf1, out_v, b_v, sem0, sem1):
    wid = lax.axis_index("s") * NC + lax.axis_index("c")

    pltpu.sync_copy(x2_hbm.at[pl.ds(wid * CHUNKS_PER_W, CHUNKS_PER_W)], idx_v)
    pltpu.sync_copy(bvec_hbm, b_v)

    def issue_group(g, buf, sem):
        base = g * STREAMS_PER_GROUP
        for k in range(STREAMS_PER_GROUP):
            pltpu.async_copy(proj_hbm.at[idx_v.at[base + k]],
                             buf.at[pl.ds(k * TOK_CHUNK, TOK_CHUNK)], sem)

    def wait_group(g, buf, sem):
        base = g * STREAMS_PER_GROUP
        for k in range(STREAMS_PER_GROUP):
            pltpu.make_async_copy(proj_hbm.at[idx_v.at[base + k]],
                                  buf.at[pl.ds(k * TOK_CHUNK, TOK_CHUNK)],
                                  sem).wait()

    def compute_group(g, buf):
        bvec = b_v[...]

        def row_body(r, carry):
            base = r * SEQ
            a0 = jnp.zeros((DP,), jnp.float32)
            a1 = jnp.zeros((DP,), jnp.float32)
            a2 = jnp.zeros((DP,), jnp.float32)
            a3 = jnp.zeros((DP,), jnp.float32)
            for j in range(0, SEQ, 4):
                a0 = a0 + buf[base + j]
                a1 = a1 + buf[base + j + 1]
                a2 = a2 + buf[base + j + 2]
                a3 = a3 + buf[base + j + 3]
            z = ((a0 + a1) + (a2 + a3)) + bvec
            out_v[g * G_ROWS + r] = 1.0 / (1.0 + jnp.exp(-z))
            return carry

        lax.fori_loop(0, G_ROWS, row_body, 0)

    issue_group(0, buf0, sem0)
    issue_group(1, buf1, sem1)

    def outer(i, carry):
        for bsel in range(2):
            g = i * 2 + bsel
            buf = buf0 if bsel == 0 else buf1
            sem = sem0 if bsel == 0 else sem1
            wait_group(g, buf, sem)
            compute_group(g, buf)

            @pl.when(g + 2 < NGROUPS)
            def _():
                issue_group(g + 2, buf, sem)
        return carry

    lax.fori_loop(0, NGROUPS // 2, outer, 0)

    pltpu.sync_copy(out_v, out_hbm.at[pl.ds(wid * ROWS_PER_W, ROWS_PER_W)])


def _pool(proj, x2, bvec):
    mesh = plsc.VectorSubcoreMesh(core_axis_name="c", subcore_axis_name="s")
    fn = pl.kernel(
        _pool_body,
        mesh=mesh,
        compiler_params=pltpu.CompilerParams(use_tc_tiling_on_sc=False),
        out_type=jax.ShapeDtypeStruct((BATCH, DP), jnp.float32),
        scratch_types=[
            pltpu.VMEM((CHUNKS_PER_W, TOK_CHUNK), jnp.int32),
            pltpu.VMEM((G_ROWS * SEQ, DP), jnp.float32),
            pltpu.VMEM((G_ROWS * SEQ, DP), jnp.float32),
            pltpu.VMEM((ROWS_PER_W, DP), jnp.float32),
            pltpu.VMEM((DP,), jnp.float32),
            pltpu.SemaphoreType.DMA,
            pltpu.SemaphoreType.DMA,
        ],
    )
    return fn(proj, x2, bvec)


def kernel(x, emb, W, b):
    x = x.astype(jnp.int32)
    wt = jnp.zeros((EMB_DIM, DP), jnp.float32).at[:, :NUM_CLASSES].set(
        W.T * (1.0 / SEQ))
    bvec = jnp.zeros((DP,), jnp.float32).at[:NUM_CLASSES].set(b)
    proj = _project(emb, wt)
    x2 = x.reshape(BATCH * SEQ // TOK_CHUNK, TOK_CHUNK)
    out = _pool(proj, x2, bvec)
    return out[:, :NUM_CLASSES]


# speedup vs baseline: 8.3254x; 1.1292x over previous
import functools

import jax
import jax.numpy as jnp
from jax import lax
from jax.experimental import pallas as pl
from jax.experimental.pallas import tpu as pltpu
from jax.experimental.pallas import tpu_sc as plsc

VOCAB = 100000
EMB_DIM = 300
NUM_CLASSES = 2
BATCH = 4096
SEQ = 200

DP = 16
NC, NS = 2, 16
NW = NC * NS
ROWS_PER_W = BATCH // NW
G_ROWS = 8
NGROUPS = ROWS_PER_W // G_ROWS
TOK_CHUNK = 100
STREAMS_PER_GROUP = G_ROWS * SEQ // TOK_CHUNK
CHUNKS_PER_W = ROWS_PER_W * SEQ // TOK_CHUNK

PROJ_BLK = 2048


NBLK = (VOCAB + PROJ_BLK - 1) // PROJ_BLK
VOCAB_PAD = NBLK * PROJ_BLK
SUB = PROJ_BLK // 8


def _proj_body(emb_ref, wt_ref, out_ref):
    res = jnp.dot(emb_ref[...], wt_ref[...],
                  preferred_element_type=jnp.float32)
    out_ref[...] = jnp.concatenate(
        [res[SUB * k:SUB * (k + 1)] for k in range(8)], axis=1)


def _project(emb, wt):
    return pl.pallas_call(
        _proj_body,
        grid=(NBLK,),
        in_specs=[
            pl.BlockSpec((PROJ_BLK, EMB_DIM), lambda i: (i, 0)),
            pl.BlockSpec((EMB_DIM, DP), lambda i: (0, 0)),
        ],
        out_specs=pl.BlockSpec((SUB, 8 * DP), lambda i: (i, 0)),
        out_shape=jax.ShapeDtypeStruct((NBLK * SUB, 8 * DP), jnp.float32),
    )(emb, wt)


def _pool_body(proj_hbm, x2_hbm, bvec_hbm, out_hbm,
               idx_v, buf0, buf1, out_v, b_v, sem0, sem1):
    wid = lax.axis_index("s") * NC + lax.axis_index("c")

    pltpu.sync_copy(x2_hbm.at[pl.ds(wid * CHUNKS_PER_W, CHUNKS_PER_W)], idx_v)
    pltpu.sync_copy(bvec_hbm, b_v)

    def issue_group(g, buf, sem):
        base = g * STREAMS_PER_GROUP
        for k in range(STREAMS_PER_GROUP):
            pltpu.async_copy(proj_hbm.at[idx_v.at[base + k]],
                             buf.at[pl.ds(k * TOK_CHUNK, TOK_CHUNK)], sem)

    def wait_group(g, buf, sem):
        base = g * STREAMS_PER_GROUP
        for k in range(STREAMS_PER_GROUP):
            pltpu.make_async_copy(proj_hbm.at[idx_v.at[base + k]],
                                  buf.at[pl.ds(k * TOK_CHUNK, TOK_CHUNK)],
                                  sem).wait()

    def compute_group(g, buf):
        bvec = b_v[...]

        def row_body(r, carry):
            base = r * SEQ
            a0 = jnp.zeros((DP,), jnp.float32)
            a1 = jnp.zeros((DP,), jnp.float32)
            a2 = jnp.zeros((DP,), jnp.float32)
            a3 = jnp.zeros((DP,), jnp.float32)
            for j in range(0, SEQ, 4):
                a0 = a0 + buf[base + j]
                a1 = a1 + buf[base + j + 1]
                a2 = a2 + buf[base + j + 2]
                a3 = a3 + buf[base + j + 3]
            z = ((a0 + a1) + (a2 + a3)) + bvec
            out_v[g * G_ROWS + r] = 1.0 / (1.0 + jnp.exp(-z))
            return carry

        lax.fori_loop(0, G_ROWS, row_body, 0)

    issue_group(0, buf0, sem0)
    issue_group(1, buf1, sem1)

    def outer(i, carry):
        for bsel in range(2):
            g = i * 2 + bsel
            buf = buf0 if bsel == 0 else buf1
            sem = sem0 if bsel == 0 else sem1
            wait_group(g, buf, sem)
            compute_group(g, buf)

            @pl.when(g + 2 < NGROUPS)
            def _():
                issue_group(g + 2, buf, sem)
        return carry

    lax.fori_loop(0, NGROUPS // 2, outer, 0)

    pltpu.sync_copy(out_v, out_hbm.at[pl.ds(wid * ROWS_PER_W, ROWS_PER_W)])


def _pool(proj, x2, bvec):
    mesh = plsc.VectorSubcoreMesh(core_axis_name="c", subcore_axis_name="s")
    fn = pl.kernel(
        _pool_body,
        mesh=mesh,
        compiler_params=pltpu.CompilerParams(use_tc_tiling_on_sc=False),
        out_type=jax.ShapeDtypeStruct((BATCH, DP), jnp.float32),
        scratch_types=[
            pltpu.VMEM((CHUNKS_PER_W, TOK_CHUNK), jnp.int32),
            pltpu.VMEM((G_ROWS * SEQ, DP), jnp.float32),
            pltpu.VMEM((G_ROWS * SEQ, DP), jnp.float32),
            pltpu.VMEM((ROWS_PER_W, DP), jnp.float32),
            pltpu.VMEM((DP,), jnp.float32),
            pltpu.SemaphoreType.DMA,
            pltpu.SemaphoreType.DMA,
        ],
    )
    return fn(proj, x2, bvec)


def kernel(x, emb, W, b):
    x = x.astype(jnp.int32)
    wt = jnp.zeros((EMB_DIM, DP), jnp.float32).at[:, :NUM_CLASSES].set(
        W.T * (1.0 / SEQ))
    bvec = jnp.zeros((DP,), jnp.float32).at[:NUM_CLASSES].set(b)
    proj = _project(emb, wt).reshape(VOCAB_PAD, DP)
    u = (x & ~2047) | ((x & 255) << 3) | ((x >> 8) & 7)
    x2 = u.reshape(BATCH * SEQ // TOK_CHUNK, TOK_CHUNK)
    out = _pool(proj, x2, bvec)
    return out[:, :NUM_CLASSES]


# speedup vs baseline: 14.9681x; 1.7979x over previous
import functools

import jax
import jax.numpy as jnp
from jax import lax
from jax.experimental import pallas as pl
from jax.experimental.pallas import tpu as pltpu
from jax.experimental.pallas import tpu_sc as plsc

VOCAB = 100000
EMB_DIM = 300
NUM_CLASSES = 2
BATCH = 4096
SEQ = 200

DP = 16
NC, NS = 2, 16
NW = NC * NS
ROWS_PER_W = BATCH // NW
G_ROWS = 8
NGROUPS = ROWS_PER_W // G_ROWS
TOK_CHUNK = 100
STREAMS_PER_GROUP = G_ROWS * SEQ // TOK_CHUNK
CHUNKS_PER_W = ROWS_PER_W * SEQ // TOK_CHUNK

PROJ_BLK = 2048


NBLK = (VOCAB + PROJ_BLK - 1) // PROJ_BLK
VOCAB_PAD = NBLK * PROJ_BLK
SUB = PROJ_BLK // 8


def _proj_body(embT_ref, wt_ref, out_ref):
    res = lax.dot_general(embT_ref[...], wt_ref[...],
                          dimension_numbers=(((0,), (0,)), ((), ())),
                          preferred_element_type=jnp.float32)
    out_ref[...] = jnp.concatenate(
        [res[SUB * k:SUB * (k + 1)] for k in range(8)], axis=1)


def _project(embT, wt):
    return pl.pallas_call(
        _proj_body,
        grid=(NBLK,),
        in_specs=[
            pl.BlockSpec((EMB_DIM, PROJ_BLK), lambda i: (0, i)),
            pl.BlockSpec((EMB_DIM, DP), lambda i: (0, 0)),
        ],
        out_specs=pl.BlockSpec((SUB, 8 * DP), lambda i: (i, 0)),
        out_shape=jax.ShapeDtypeStruct((NBLK * SUB, 8 * DP), jnp.float32),
    )(embT, wt)


def _pool_body(proj_hbm, x2_hbm, bvec_hbm, out_hbm,
               idx_v, buf0, buf1, out_v, b_v, sem0, sem1):
    wid = lax.axis_index("s") * NC + lax.axis_index("c")

    pltpu.sync_copy(x2_hbm.at[pl.ds(wid * CHUNKS_PER_W, CHUNKS_PER_W)], idx_v)
    pltpu.sync_copy(bvec_hbm, b_v)

    def issue_group(g, buf, sem):
        base = g * STREAMS_PER_GROUP
        for k in range(STREAMS_PER_GROUP):
            pltpu.async_copy(proj_hbm.at[idx_v.at[base + k]],
                             buf.at[pl.ds(k * TOK_CHUNK, TOK_CHUNK)], sem)

    def wait_group(g, buf, sem):
        base = g * STREAMS_PER_GROUP
        for k in range(STREAMS_PER_GROUP):
            pltpu.make_async_copy(proj_hbm.at[idx_v.at[base + k]],
                                  buf.at[pl.ds(k * TOK_CHUNK, TOK_CHUNK)],
                                  sem).wait()

    def compute_group(g, buf):
        bvec = b_v[...]

        def row_body(r, carry):
            base = r * SEQ
            a0 = jnp.zeros((DP,), jnp.float32)
            a1 = jnp.zeros((DP,), jnp.float32)
            a2 = jnp.zeros((DP,), jnp.float32)
            a3 = jnp.zeros((DP,), jnp.float32)
            for j in range(0, SEQ, 4):
                a0 = a0 + buf[base + j]
                a1 = a1 + buf[base + j + 1]
                a2 = a2 + buf[base + j + 2]
                a3 = a3 + buf[base + j + 3]
            z = ((a0 + a1) + (a2 + a3)) + bvec
            out_v[g * G_ROWS + r] = 1.0 / (1.0 + jnp.exp(-z))
            return carry

        lax.fori_loop(0, G_ROWS, row_body, 0)

    issue_group(0, buf0, sem0)
    issue_group(1, buf1, sem1)

    def outer(i, carry):
        for bsel in range(2):
            g = i * 2 + bsel
            buf = buf0 if bsel == 0 else buf1
            sem = sem0 if bsel == 0 else sem1
            wait_group(g, buf, sem)
            compute_group(g, buf)

            @pl.when(g + 2 < NGROUPS)
            def _():
                issue_group(g + 2, buf, sem)
        return carry

    lax.fori_loop(0, NGROUPS // 2, outer, 0)

    pltpu.sync_copy(out_v, out_hbm.at[pl.ds(wid * ROWS_PER_W, ROWS_PER_W)])


def _pool(proj, x2, bvec):
    mesh = plsc.VectorSubcoreMesh(core_axis_name="c", subcore_axis_name="s")
    fn = pl.kernel(
        _pool_body,
        mesh=mesh,
        compiler_params=pltpu.CompilerParams(use_tc_tiling_on_sc=False),
        out_type=jax.ShapeDtypeStruct((BATCH, DP), jnp.float32),
        scratch_types=[
            pltpu.VMEM((CHUNKS_PER_W, TOK_CHUNK), jnp.int32),
            pltpu.VMEM((G_ROWS * SEQ, DP), jnp.float32),
            pltpu.VMEM((G_ROWS * SEQ, DP), jnp.float32),
            pltpu.VMEM((ROWS_PER_W, DP), jnp.float32),
            pltpu.VMEM((DP,), jnp.float32),
            pltpu.SemaphoreType.DMA,
            pltpu.SemaphoreType.DMA,
        ],
    )
    return fn(proj, x2, bvec)


def kernel(x, emb, W, b):
    x = x.astype(jnp.int32)
    wt = jnp.zeros((EMB_DIM, DP), jnp.float32).at[:, :NUM_CLASSES].set(
        W.T * (1.0 / SEQ))
    bvec = jnp.zeros((DP,), jnp.float32).at[:NUM_CLASSES].set(b)
    proj = _project(emb.T, wt).reshape(VOCAB_PAD, DP)
    u = (x & ~2047) | ((x & 255) << 3) | ((x >> 8) & 7)
    x2 = u.reshape(BATCH * SEQ // TOK_CHUNK, TOK_CHUNK)
    out = _pool(proj, x2, bvec)
    return out[:, :NUM_CLASSES]
